# baseline (speedup 1.0000x reference)
import jax
import jax.numpy as jnp
from jax.experimental import pallas as pl

_CH = 32


def _body(o_ref):
    o_ref[...] = jnp.zeros_like(o_ref)


def kernel(x_start, t, noise, sqrt_alphas_cumprod, sqrt_one_minus_alphas_cumprod):
    B, C, H, W = x_start.shape
    out = pl.pallas_call(
        _body,
        grid=(B // _CH,),
        out_specs=pl.BlockSpec((_CH, C, H, W), lambda i: (i, 0, 0, 0)),
        out_shape=jax.ShapeDtypeStruct((B, C, H, W), jnp.float32),
    )()
    return out


# speedup vs baseline: 1.5471x; 1.5471x over previous
import jax
import jax.numpy as jnp
from jax.experimental import pallas as pl
from jax.experimental.pallas import tpu as pltpu

_FR = 1536
_TPAD = 1024


def _body(t_ref, sac_ref, somac_ref, x_ref, n_ref, o_ref, c1v, c2v):
    @pl.when(pl.program_id(0) == 0)
    def _():
        B = t_ref.shape[1]
        sub = jax.lax.broadcasted_iota(jnp.int32, (_TPAD, B), 0)
        hot = sub == t_ref[...]
        zero = jnp.zeros((_TPAD, B), jnp.float32)
        c1v[...] = jnp.sum(jnp.where(hot, sac_ref[...], zero), axis=0,
                           keepdims=True)
        c2v[...] = jnp.sum(jnp.where(hot, somac_ref[...], zero), axis=0,
                           keepdims=True)

    o_ref[...] = c1v[...] * x_ref[...] + c2v[...] * n_ref[...]


def kernel(x_start, t, noise, sqrt_alphas_cumprod, sqrt_one_minus_alphas_cumprod):
    B, C, H, W = x_start.shape
    F = C * H * W
    xt = x_start.transpose(1, 2, 3, 0).reshape(F, B)
    nt = noise.transpose(1, 2, 3, 0).reshape(F, B)
    t2 = t.reshape(1, B)
    sac = jnp.pad(
        sqrt_alphas_cumprod, (0, _TPAD - sqrt_alphas_cumprod.shape[0])
    ).reshape(_TPAD, 1)
    somac = jnp.pad(
        sqrt_one_minus_alphas_cumprod,
        (0, _TPAD - sqrt_one_minus_alphas_cumprod.shape[0]),
    ).reshape(_TPAD, 1)

    out = pl.pallas_call(
        _body,
        grid=(F // _FR,),
        in_specs=[
            pl.BlockSpec((1, B), lambda i: (0, 0)),
            pl.BlockSpec((_TPAD, 1), lambda i: (0, 0)),
            pl.BlockSpec((_TPAD, 1), lambda i: (0, 0)),
            pl.BlockSpec((_FR, B), lambda i: (i, 0)),
            pl.BlockSpec((_FR, B), lambda i: (i, 0)),
        ],
        out_specs=pl.BlockSpec((_FR, B), lambda i: (i, 0)),
        out_shape=jax.ShapeDtypeStruct((F, B), jnp.float32),
        scratch_shapes=[
            pltpu.VMEM((1, B), jnp.float32),
            pltpu.VMEM((1, B), jnp.float32),
        ],
    )(t2, sac, somac, xt, nt)
    return out.reshape(C, H, W, B).transpose(3, 0, 1, 2)
